# Initial kernel scaffold; baseline (speedup 1.0000x reference)
import jax
import jax.numpy as jnp
from jax.experimental import pallas as pl


def kernel(token_ids, embedding):
    raise NotImplementedError("write your pallas kernel here")


# speedup vs baseline: 5.2074x; 5.2074x over previous
import functools

import jax
import jax.numpy as jnp
from jax import lax
from jax.experimental import pallas as pl
from jax.experimental.pallas import tpu as pltpu
from jax.experimental.pallas import tpu_sc as plsc

_D = 64
_B = 16384 * 50
_NC = 2
_NS = 16
_NW = _NC * _NS
_RPW = _B // _NW
_CH = 128
_NCH = _RPW // _CH


def _make_gather():
    mesh = plsc.VectorSubcoreMesh(core_axis_name="c", subcore_axis_name="s")

    @functools.partial(
        pl.kernel,
        mesh=mesh,
        out_type=jax.ShapeDtypeStruct((_B, _D), jnp.float32),
        scratch_types=[
            pltpu.VMEM((_RPW,), jnp.int32),
            pltpu.VMEM((_CH, _D), jnp.float32),
            pltpu.SemaphoreType.DMA,
        ],
        compiler_params=pltpu.CompilerParams(use_tc_tiling_on_sc=False),
    )
    def gather_kernel(idx_hbm, table_hbm, out_hbm, idx_v, rows_v, sem):
        wid = lax.axis_index("s") * _NC + lax.axis_index("c")
        base = wid * _RPW
        pltpu.sync_copy(idx_hbm.at[pl.ds(base, _RPW)], idx_v)

        def body(i, carry):
            off = i * _CH
            pltpu.async_copy(
                table_hbm.at[idx_v.at[pl.ds(off, _CH)]], rows_v, sem
            ).wait()
            pltpu.sync_copy(rows_v, out_hbm.at[pl.ds(base + off, _CH)])
            return carry

        lax.fori_loop(0, _NCH, body, 0)

    return gather_kernel


_gather = _make_gather()


def kernel(token_ids, embedding):
    idx = token_ids.reshape(-1)
    out = _gather(idx, embedding)
    return out.reshape(*token_ids.shape, embedding.shape[1])


# speedup vs baseline: 6.2083x; 1.1922x over previous
import functools

import jax
import jax.numpy as jnp
from jax import lax
from jax.experimental import pallas as pl
from jax.experimental.pallas import tpu as pltpu
from jax.experimental.pallas import tpu_sc as plsc

_D = 64
_B = 16384 * 50
_NC = 2
_NS = 16
_NW = _NC * _NS
_RPW = _B // _NW
_CH = 128
_NCH = _RPW // _CH
_NBUF = 8
_NGRP = _NCH // _NBUF


def _make_gather():
    mesh = plsc.VectorSubcoreMesh(core_axis_name="c", subcore_axis_name="s")

    @functools.partial(
        pl.kernel,
        mesh=mesh,
        out_type=jax.ShapeDtypeStruct((_B, _D), jnp.float32),
        scratch_types=[
            pltpu.VMEM((_RPW,), jnp.int32),
            pltpu.VMEM((_NBUF * _CH, _D), jnp.float32),
            pltpu.SemaphoreType.DMA((_NBUF,)),
            pltpu.SemaphoreType.DMA((_NBUF,)),
        ],
        compiler_params=pltpu.CompilerParams(use_tc_tiling_on_sc=False),
    )
    def gather_kernel(idx_hbm, table_hbm, out_hbm, idx_v, rows_v, gsem, ssem):
        wid = lax.axis_index("s") * _NC + lax.axis_index("c")
        base = wid * _RPW
        pltpu.sync_copy(idx_hbm.at[pl.ds(base, _RPW)], idx_v)

        def start_gather(i, b):
            pltpu.make_async_copy(
                table_hbm.at[idx_v.at[pl.ds(i * _CH, _CH)]],
                rows_v.at[pl.ds(b * _CH, _CH)],
                gsem.at[b],
            ).start()

        def wait_gather(i, b):
            pltpu.make_async_copy(
                table_hbm.at[idx_v.at[pl.ds(i * _CH, _CH)]],
                rows_v.at[pl.ds(b * _CH, _CH)],
                gsem.at[b],
            ).wait()

        def start_store(i, b):
            pltpu.make_async_copy(
                rows_v.at[pl.ds(b * _CH, _CH)],
                out_hbm.at[pl.ds(base + i * _CH, _CH)],
                ssem.at[b],
            ).start()

        def wait_store(i, b):
            pltpu.make_async_copy(
                rows_v.at[pl.ds(b * _CH, _CH)],
                out_hbm.at[pl.ds(base + i * _CH, _CH)],
                ssem.at[b],
            ).wait()

        for b in range(_NBUF):
            start_gather(b, b)

        def body(g, carry):
            i0 = g * _NBUF
            for b in range(_NBUF):
                wait_gather(i0 + b, b)
                start_store(i0 + b, b)
            @pl.when(g < _NGRP - 1)
            def _():
                for b in range(_NBUF):
                    wait_store(i0 + b, b)
                    start_gather(i0 + _NBUF + b, b)

            return carry

        lax.fori_loop(0, _NGRP, body, 0)

        i0 = (_NGRP - 1) * _NBUF
        for b in range(_NBUF):
            wait_store(i0 + b, b)

    return gather_kernel


_gather = _make_gather()


def kernel(token_ids, embedding):
    idx = token_ids.reshape(-1)
    out = _gather(idx, embedding)
    return out.reshape(*token_ids.shape, embedding.shape[1])
